# baseline (speedup 1.0000x reference)
import functools

import jax
import jax.numpy as jnp
from jax import lax
from jax.experimental import pallas as pl
from jax.experimental.pallas import tpu as pltpu
from jax.experimental.pallas import tpu_sc as plsc

NBUF = 4
LOOKAHEAD = 2


def _sc_geometry():
    try:
        info = plsc.get_sparse_core_info()
        return info.num_cores, info.num_subcores
    except Exception:
        return 2, 16


@functools.cache
def _build(B, L, D):
    NC, NS = _sc_geometry()
    NW = NC * NS
    assert B % (NW * NBUF) == 0 and L <= 128
    n_rows = B // NW
    mesh = plsc.VectorSubcoreMesh(core_axis_name="c", subcore_axis_name="s")

    @functools.partial(
        pl.kernel,
        out_type=jax.ShapeDtypeStruct((B, L, D), jnp.float32),
        mesh=mesh,
        compiler_params=pltpu.CompilerParams(use_tc_tiling_on_sc=True),
        scratch_types=[
            pltpu.VMEM((n_rows, L), jnp.int32),
            [pltpu.VMEM((L, D), jnp.float32) for _ in range(NBUF)],
            [pltpu.SemaphoreType.DMA for _ in range(NBUF)],
            [pltpu.SemaphoreType.DMA for _ in range(NBUF)],
        ],
    )
    def emb(idx_hbm, table_hbm, out_hbm, idx_v, rows, gsem, wsem):
        wid = lax.axis_index("s") * NC + lax.axis_index("c")
        base = wid * n_rows
        pltpu.sync_copy(idx_hbm.at[pl.ds(base, n_rows)], idx_v)

        def start_gather(r, b):
            pltpu.async_copy(table_hbm.at[idx_v.at[r]], rows[b], gsem[b])

        def start_write(r, b):
            pltpu.async_copy(rows[b], out_hbm.at[base + r], wsem[b])

        def wait_write(b):
            pltpu.make_async_copy(rows[b], out_hbm.at[0], wsem[b]).wait()

        def wait_gather(r, b):
            pltpu.make_async_copy(
                table_hbm.at[idx_v.at[r]], rows[b], gsem[b]
            ).wait()

        for p in range(LOOKAHEAD):
            start_gather(p, p)

        @pl.loop(0, n_rows, step=NBUF)
        def _(j0):
            for b in range(NBUF):
                j = j0 + b
                g = j + LOOKAHEAD
                bg = (b + LOOKAHEAD) % NBUF

                @pl.when(g < n_rows)
                def _():
                    @pl.when(j >= NBUF - LOOKAHEAD)
                    def _():
                        wait_write(bg)

                    start_gather(g, bg)

                wait_gather(j, b)
                start_write(j, b)

        for b in range(NBUF):
            wait_write(b)

    return emb


def kernel(token_ids, weight):
    B, L = token_ids.shape
    D = weight.shape[1]
    return _build(B, L, D)(token_ids, weight)


# speedup vs baseline: 1.7733x; 1.7733x over previous
import functools

import jax
import jax.numpy as jnp
from jax import lax
from jax.experimental import pallas as pl
from jax.experimental.pallas import tpu as pltpu
from jax.experimental.pallas import tpu_sc as plsc

CHUNK = 128
NBUF = 5
LOOKAHEAD = 2


def _sc_geometry():
    try:
        info = plsc.get_sparse_core_info()
        return info.num_cores, info.num_subcores
    except Exception:
        return 2, 16


@functools.cache
def _build(N, D):
    NC, NS = _sc_geometry()
    NW = NC * NS
    assert N % (NW * CHUNK) == 0
    n_chunks = N // (NW * CHUNK)
    assert n_chunks % NBUF == 0
    mesh = plsc.VectorSubcoreMesh(core_axis_name="c", subcore_axis_name="s")

    @functools.partial(
        pl.kernel,
        out_type=jax.ShapeDtypeStruct((N, D), jnp.float32),
        mesh=mesh,
        scratch_types=[
            pltpu.VMEM((n_chunks, CHUNK), jnp.int32),
            [pltpu.VMEM((CHUNK, D), jnp.float32) for _ in range(NBUF)],
            [pltpu.SemaphoreType.DMA for _ in range(NBUF)],
            [pltpu.SemaphoreType.DMA for _ in range(NBUF)],
        ],
    )
    def emb(idx_hbm, table_hbm, out_hbm, idx_v, rows, gsem, wsem):
        wid = lax.axis_index("s") * NC + lax.axis_index("c")
        base_chunk = wid * n_chunks
        pltpu.sync_copy(idx_hbm.at[wid], idx_v)

        def start_gather(chunk, b):
            pltpu.async_copy(table_hbm.at[idx_v.at[chunk]], rows[b], gsem[b])

        def start_write(chunk, b):
            pltpu.async_copy(
                rows[b],
                out_hbm.at[pl.ds((base_chunk + chunk) * CHUNK, CHUNK)],
                wsem[b],
            )

        def wait_write(b):
            pltpu.make_async_copy(
                rows[b], out_hbm.at[pl.ds(0, CHUNK)], wsem[b]
            ).wait()

        def wait_gather(chunk, b):
            pltpu.make_async_copy(
                table_hbm.at[idx_v.at[chunk]], rows[b], gsem[b]
            ).wait()

        for p in range(LOOKAHEAD):
            start_gather(p, p)

        @pl.loop(0, n_chunks, step=NBUF)
        def _(j0):
            for b in range(NBUF):
                j = j0 + b
                g = j + LOOKAHEAD
                bg = (b + LOOKAHEAD) % NBUF

                @pl.when(g < n_chunks)
                def _():
                    @pl.when(j >= NBUF - LOOKAHEAD)
                    def _():
                        wait_write(bg)

                    start_gather(g, bg)

                wait_gather(j, b)
                start_write(j, b)

        for b in range(NBUF):
            wait_write(b)

    return emb


def kernel(token_ids, weight):
    B, L = token_ids.shape
    D = weight.shape[1]
    N = B * L
    NC, NS = _sc_geometry()
    NW = NC * NS
    idx = token_ids.T.reshape(NW, N // (NW * CHUNK), CHUNK)
    out = _build(N, D)(idx, weight)
    return out.reshape(L, B, D).transpose(1, 0, 2)


# speedup vs baseline: 1.7735x; 1.0002x over previous
import functools

import jax
import jax.numpy as jnp
from jax import lax
from jax.experimental import pallas as pl
from jax.experimental.pallas import tpu as pltpu
from jax.experimental.pallas import tpu_sc as plsc

CHUNK = 128
NBUF = 5
LOOKAHEAD = 3


def _sc_geometry():
    try:
        info = plsc.get_sparse_core_info()
        return info.num_cores, info.num_subcores
    except Exception:
        return 2, 16


@functools.cache
def _build(N, D):
    NC, NS = _sc_geometry()
    NW = NC * NS
    assert N % (NW * CHUNK) == 0
    n_chunks = N // (NW * CHUNK)
    assert n_chunks % NBUF == 0
    mesh = plsc.VectorSubcoreMesh(core_axis_name="c", subcore_axis_name="s")

    @functools.partial(
        pl.kernel,
        out_type=jax.ShapeDtypeStruct((N, D), jnp.float32),
        mesh=mesh,
        scratch_types=[
            pltpu.VMEM((n_chunks, CHUNK), jnp.int32),
            [pltpu.VMEM((CHUNK, D), jnp.float32) for _ in range(NBUF)],
            [pltpu.SemaphoreType.DMA for _ in range(NBUF)],
            [pltpu.SemaphoreType.DMA for _ in range(NBUF)],
        ],
    )
    def emb(idx_hbm, table_hbm, out_hbm, idx_v, rows, gsem, wsem):
        wid = lax.axis_index("s") * NC + lax.axis_index("c")
        base_chunk = wid * n_chunks
        pltpu.sync_copy(idx_hbm.at[wid], idx_v)

        def start_gather(chunk, b):
            pltpu.async_copy(table_hbm.at[idx_v.at[chunk]], rows[b], gsem[b])

        def start_write(chunk, b):
            pltpu.async_copy(
                rows[b],
                out_hbm.at[pl.ds((base_chunk + chunk) * CHUNK, CHUNK)],
                wsem[b],
            )

        def wait_write(b):
            pltpu.make_async_copy(
                rows[b], out_hbm.at[pl.ds(0, CHUNK)], wsem[b]
            ).wait()

        def wait_gather(chunk, b):
            pltpu.make_async_copy(
                table_hbm.at[idx_v.at[chunk]], rows[b], gsem[b]
            ).wait()

        for p in range(LOOKAHEAD):
            start_gather(p, p)

        @pl.loop(0, n_chunks, step=NBUF)
        def _(j0):
            for b in range(NBUF):
                j = j0 + b
                g = j + LOOKAHEAD
                bg = (b + LOOKAHEAD) % NBUF

                @pl.when(g < n_chunks)
                def _():
                    @pl.when(j >= NBUF - LOOKAHEAD)
                    def _():
                        wait_write(bg)

                    start_gather(g, bg)

                wait_gather(j, b)
                start_write(j, b)

        for b in range(NBUF):
            wait_write(b)

    return emb


def kernel(token_ids, weight):
    B, L = token_ids.shape
    D = weight.shape[1]
    N = B * L
    NC, NS = _sc_geometry()
    NW = NC * NS
    idx = token_ids.T.reshape(NW, N // (NW * CHUNK), CHUNK)
    out = _build(N, D)(idx, weight)
    return out.reshape(L, B, D).transpose(1, 0, 2)


# speedup vs baseline: 1.7736x; 1.0000x over previous
import functools

import jax
import jax.numpy as jnp
from jax import lax
from jax.experimental import pallas as pl
from jax.experimental.pallas import tpu as pltpu
from jax.experimental.pallas import tpu_sc as plsc

CHUNK = 128
NBUF = 5
LOOKAHEAD = 3


def _sc_geometry():
    try:
        info = plsc.get_sparse_core_info()
        return info.num_cores, info.num_subcores
    except Exception:
        return 2, 16


@functools.cache
def _build(N, D):
    NC, NS = _sc_geometry()
    NW = NC * NS
    assert N % (NW * CHUNK) == 0
    n_chunks = N // (NW * CHUNK)
    assert n_chunks % NBUF == 0
    mesh = plsc.VectorSubcoreMesh(core_axis_name="c", subcore_axis_name="s")

    @functools.partial(
        pl.kernel,
        out_type=jax.ShapeDtypeStruct((N, D), jnp.float32),
        mesh=mesh,
        scratch_types=[
            pltpu.VMEM((n_chunks * CHUNK,), jnp.int32),
            [pltpu.VMEM((CHUNK, D), jnp.float32) for _ in range(NBUF)],
            [pltpu.SemaphoreType.DMA for _ in range(NBUF)],
            [pltpu.SemaphoreType.DMA for _ in range(NBUF)],
        ],
    )
    def emb(idx_hbm, table_hbm, out_hbm, idx_v, rows, gsem, wsem):
        wid = lax.axis_index("s") * NC + lax.axis_index("c")
        base_chunk = wid * n_chunks
        pltpu.sync_copy(idx_hbm.at[wid], idx_v)

        def idx_slice(chunk):
            return idx_v.at[pl.ds(chunk * CHUNK, CHUNK)]

        def start_gather(chunk, b):
            pltpu.async_copy(table_hbm.at[idx_slice(chunk)], rows[b], gsem[b])

        def start_write(chunk, b):
            pltpu.async_copy(
                rows[b],
                out_hbm.at[pl.ds((base_chunk + chunk) * CHUNK, CHUNK)],
                wsem[b],
            )

        def wait_write(b):
            pltpu.make_async_copy(
                rows[b], out_hbm.at[pl.ds(0, CHUNK)], wsem[b]
            ).wait()

        def wait_gather(chunk, b):
            pltpu.make_async_copy(
                table_hbm.at[idx_slice(chunk)], rows[b], gsem[b]
            ).wait()

        for p in range(LOOKAHEAD):
            start_gather(p, p)

        @pl.loop(0, n_chunks, step=NBUF)
        def _(j0):
            for b in range(NBUF):
                j = j0 + b
                g = j + LOOKAHEAD
                bg = (b + LOOKAHEAD) % NBUF

                @pl.when(g < n_chunks)
                def _():
                    @pl.when(j >= NBUF - LOOKAHEAD)
                    def _():
                        wait_write(bg)

                    start_gather(g, bg)

                wait_gather(j, b)
                start_write(j, b)

        for b in range(NBUF):
            wait_write(b)

    return emb


def kernel(token_ids, weight):
    B, L = token_ids.shape
    D = weight.shape[1]
    N = B * L
    NC, NS = _sc_geometry()
    NW = NC * NS
    idx = token_ids.T.reshape(NW, N // NW)
    out = _build(N, D)(idx, weight)
    return out.reshape(L, B, D).transpose(1, 0, 2)


# speedup vs baseline: 1.8215x; 1.0270x over previous
import functools

import jax
import jax.numpy as jnp
from jax import lax
from jax.experimental import pallas as pl
from jax.experimental.pallas import tpu as pltpu
from jax.experimental.pallas import tpu_sc as plsc

CHUNK = 128
NBUF = 5
LOOKAHEAD = 3


def _sc_geometry():
    try:
        info = plsc.get_sparse_core_info()
        return info.num_cores, info.num_subcores
    except Exception:
        return 2, 16


@functools.cache
def _build(L, B, D):
    NC, NS = _sc_geometry()
    NW = NC * NS
    assert B % (NW * CHUNK) == 0
    n_chunks = L
    assert n_chunks % NBUF == 0
    mesh = plsc.VectorSubcoreMesh(core_axis_name="c", subcore_axis_name="s")

    @functools.partial(
        pl.kernel,
        out_type=jax.ShapeDtypeStruct((L * B, D), jnp.float32),
        mesh=mesh,
        scratch_types=[
            pltpu.VMEM((n_chunks, CHUNK), jnp.int32),
            [pltpu.VMEM((CHUNK, D), jnp.float32) for _ in range(NBUF)],
            [pltpu.SemaphoreType.DMA for _ in range(NBUF)],
            [pltpu.SemaphoreType.DMA for _ in range(NBUF)],
        ],
    )
    def emb(idx_hbm, table_hbm, out_hbm, idx_v, rows, gsem, wsem):
        wid = lax.axis_index("s") * NC + lax.axis_index("c")
        col0 = wid * CHUNK
        pltpu.sync_copy(
            idx_hbm.at[pl.ds(0, n_chunks), pl.ds(col0, CHUNK)], idx_v
        )

        def start_gather(chunk, b):
            pltpu.async_copy(table_hbm.at[idx_v.at[chunk]], rows[b], gsem[b])

        def start_write(chunk, b):
            pltpu.async_copy(
                rows[b],
                out_hbm.at[pl.ds(chunk * B + col0, CHUNK)],
                wsem[b],
            )

        def wait_write(b):
            pltpu.make_async_copy(
                rows[b], out_hbm.at[pl.ds(0, CHUNK)], wsem[b]
            ).wait()

        def wait_gather(chunk, b):
            pltpu.make_async_copy(
                table_hbm.at[idx_v.at[chunk]], rows[b], gsem[b]
            ).wait()

        for p in range(LOOKAHEAD):
            start_gather(p, p)

        @pl.loop(0, n_chunks, step=NBUF)
        def _(j0):
            for b in range(NBUF):
                j = j0 + b
                g = j + LOOKAHEAD
                bg = (b + LOOKAHEAD) % NBUF

                @pl.when(g < n_chunks)
                def _():
                    @pl.when(j >= NBUF - LOOKAHEAD)
                    def _():
                        wait_write(bg)

                    start_gather(g, bg)

                wait_gather(j, b)
                start_write(j, b)

        for b in range(NBUF):
            wait_write(b)

    return emb


def kernel(token_ids, weight):
    B, L = token_ids.shape
    D = weight.shape[1]
    out = _build(L, B, D)(token_ids.T, weight)
    return out.reshape(L, B, D).transpose(1, 0, 2)


# speedup vs baseline: 1.8374x; 1.0087x over previous
import functools

import jax
import jax.numpy as jnp
from jax import lax
from jax.experimental import pallas as pl
from jax.experimental.pallas import tpu as pltpu
from jax.experimental.pallas import tpu_sc as plsc

CHUNK = 128
NBUF = 5
LOOKAHEAD = 3


def _sc_geometry():
    try:
        info = plsc.get_sparse_core_info()
        return info.num_cores, info.num_subcores
    except Exception:
        return 2, 16


@functools.cache
def _build(L, B, D):
    NC, NS = _sc_geometry()
    NW = NC * NS
    assert B % (NW * CHUNK) == 0
    n_chunks = L
    assert n_chunks % NBUF == 0
    mesh = plsc.VectorSubcoreMesh(core_axis_name="c", subcore_axis_name="s")

    @functools.partial(
        pl.kernel,
        out_type=jax.ShapeDtypeStruct((L * B, D), jnp.float32),
        mesh=mesh,
        scratch_types=[
            pltpu.VMEM((n_chunks, CHUNK), jnp.int32),
            [pltpu.VMEM((CHUNK, D), jnp.float32) for _ in range(NBUF)],
            [pltpu.SemaphoreType.DMA for _ in range(NBUF)],
            [pltpu.SemaphoreType.DMA for _ in range(NBUF)],
        ],
    )
    def emb(idx_hbm, table_hbm, out_hbm, idx_v, rows, gsem, wsem):
        wid = lax.axis_index("s") * NC + lax.axis_index("c")
        col0 = wid * CHUNK
        HEAD = 8

        def start_gather(chunk, b):
            pltpu.async_copy(table_hbm.at[idx_v.at[chunk]], rows[b], gsem[b])

        def start_write(chunk, b):
            pltpu.async_copy(
                rows[b],
                out_hbm.at[pl.ds(chunk * B + col0, CHUNK)],
                wsem[b],
            )

        def wait_write(b):
            pltpu.make_async_copy(
                rows[b], out_hbm.at[pl.ds(0, CHUNK)], wsem[b]
            ).wait()

        def wait_gather(chunk, b):
            pltpu.make_async_copy(
                table_hbm.at[idx_v.at[chunk]], rows[b], gsem[b]
            ).wait()

        pltpu.sync_copy(
            idx_hbm.at[pl.ds(0, HEAD), pl.ds(col0, CHUNK)],
            idx_v.at[pl.ds(0, HEAD)],
        )
        for p in range(LOOKAHEAD):
            start_gather(p, p)
        pltpu.sync_copy(
            idx_hbm.at[pl.ds(HEAD, n_chunks - HEAD), pl.ds(col0, CHUNK)],
            idx_v.at[pl.ds(HEAD, n_chunks - HEAD)],
        )

        @pl.loop(0, n_chunks, step=NBUF)
        def _(j0):
            for b in range(NBUF):
                j = j0 + b
                g = j + LOOKAHEAD
                bg = (b + LOOKAHEAD) % NBUF

                @pl.when(g < n_chunks)
                def _():
                    @pl.when(j >= NBUF - LOOKAHEAD)
                    def _():
                        wait_write(bg)

                    start_gather(g, bg)

                wait_gather(j, b)
                start_write(j, b)

        for b in range(NBUF):
            wait_write(b)

    return emb


def kernel(token_ids, weight):
    B, L = token_ids.shape
    D = weight.shape[1]
    out = _build(L, B, D)(token_ids.T, weight)
    return out.reshape(L, B, D).transpose(1, 0, 2)
